# baseline (speedup 1.0000x reference)
import functools

import jax
import jax.numpy as jnp
from jax import lax
from jax.experimental import pallas as pl
from jax.experimental.pallas import tpu as pltpu
from jax.experimental.pallas import tpu_sc as plsc

N = 10000
E = 320000
E2 = E // 2
DF = 128
DE = 16
H = 128
G = 64
OUT = 64

NUM_SC = 2
NUM_TILES = 16
NW = NUM_SC * NUM_TILES
PAIRS_PER_W = E2 // NW
CHUNK = 40
CHUNKS_PER_W = PAIRS_PER_W // CHUNK
ROWS_PER_TILE = 624
TAIL_ROWS = N - ROWS_PER_TILE * NUM_TILES

E8 = E2 // 8
EB = 6400
EBR = EB // 8
EGRID = E2 // EB


def _encode_body(ea_lo_ref, ea_hi_ref, wbd_ref, bbd_ref, x_ref, wn_ref,
                 bn_ref, e_ref, h_ref):
    i = pl.program_id(0)
    wbd = wbd_ref[...].astype(jnp.bfloat16)
    u_lo = (
        jnp.dot(ea_lo_ref[...].astype(jnp.bfloat16), wbd,
                preferred_element_type=jnp.float32)
        + bbd_ref[...]
    )
    u_hi = (
        jnp.dot(ea_hi_ref[...].astype(jnp.bfloat16), wbd,
                preferred_element_type=jnp.float32)
        + bbd_ref[...]
    )
    bl = lax.bitcast_convert_type(u_lo, jnp.uint32)
    bh = lax.bitcast_convert_type(u_hi, jnp.uint32)
    lo16 = lax.shift_right_logical(bl + jnp.uint32(0x8000), jnp.uint32(16))
    hi16 = (bh + jnp.uint32(0x8000)) & jnp.uint32(0xFFFF0000)
    epk = lax.bitcast_convert_type(lo16 | hi16, jnp.int32)
    for j in range(8):
        e_ref[:, j, :] = epk[:, j * H:(j + 1) * H]

    @pl.when(i == 0)
    def _():
        h_ref[...] = (
            jnp.dot(x_ref[...], wn_ref[...], preferred_element_type=jnp.float32)
            + bn_ref[...]
        )


def _tc_encode(ea8, W_bd, b_bd, x, W_node, b_node):
    return pl.pallas_call(
        _encode_body,
        grid=(EGRID,),
        in_specs=[
            pl.BlockSpec((EBR, DF), lambda i: (i, 0)),
            pl.BlockSpec((EBR, DF), lambda i: (i + EGRID, 0)),
            pl.BlockSpec((DF, 8 * H), lambda i: (0, 0)),
            pl.BlockSpec((1, 8 * H), lambda i: (0, 0)),
            pl.BlockSpec((N, DF), lambda i: (0, 0)),
            pl.BlockSpec((DF, H), lambda i: (0, 0)),
            pl.BlockSpec((1, H), lambda i: (0, 0)),
        ],
        out_specs=[
            pl.BlockSpec((EBR, 8, H), lambda i: (i, 0, 0)),
            pl.BlockSpec((N, H), lambda i: (0, 0)),
        ],
        out_shape=[
            jax.ShapeDtypeStruct((E8, 8, H), jnp.int32),
            jax.ShapeDtypeStruct((N, H), jnp.float32),
        ],
    )(ea8, ea8, W_bd, b_bd, x, W_node, b_node)


def _sc_edge_body(h_hbm, e_hbm, src_hbm, dst_hbm, zeros_hbm, out_hbm,
                  si0, si1, si2, di0, di1, di2,
                  hm0, hm1, hm2, ev0, ev1, ev2,
                  agg_sh,
                  xssem0, xssem1, xssem2, xdsem0, xdsem1, xdsem2,
                  gsem0, gsem1, gsem2, esem0, esem1, esem2,
                  ssem0, ssem1, ssem2):
    cid = lax.axis_index("c")
    sid = lax.axis_index("s")
    wid = sid * NUM_SC + cid

    sidx = (si0, si1, si2)
    didx = (di0, di1, di2)
    hm = (hm0, hm1, hm2)
    ev = (ev0, ev1, ev2)
    xssem = (xssem0, xssem1, xssem2)
    xdsem = (xdsem0, xdsem1, xdsem2)
    gsem = (gsem0, gsem1, gsem2)
    esem = (esem0, esem1, esem2)
    ssem = (ssem0, ssem1, ssem2)

    r0 = sid * ROWS_PER_TILE
    pltpu.sync_copy(zeros_hbm.at[pl.ds(r0, ROWS_PER_TILE)],
                    agg_sh.at[pl.ds(r0, ROWS_PER_TILE)])

    @pl.when(sid == NUM_TILES - 1)
    def _():
        t0 = ROWS_PER_TILE * NUM_TILES
        pltpu.sync_copy(zeros_hbm.at[pl.ds(t0, TAIL_ROWS)],
                        agg_sh.at[pl.ds(t0, TAIL_ROWS)])

    base0 = wid * PAIRS_PER_W
    ibase = wid * CHUNKS_PER_W * 2 * CHUNK

    def issue_sidx(i, b):
        pltpu.async_copy(src_hbm.at[pl.ds(ibase + i * 2 * CHUNK, 2 * CHUNK)],
                         sidx[b], xssem[b])

    def wait_sidx(b):
        pltpu.make_async_copy(src_hbm.at[pl.ds(0, 2 * CHUNK)],
                              sidx[b], xssem[b]).wait()

    def issue_didx(i, b):
        pltpu.async_copy(dst_hbm.at[pl.ds(ibase + i * 2 * CHUNK, 2 * CHUNK)],
                         didx[b], xdsem[b])

    def wait_didx(b):
        pltpu.make_async_copy(dst_hbm.at[pl.ds(0, 2 * CHUNK)],
                              didx[b], xdsem[b]).wait()

    def issue_in(i, b):
        pltpu.async_copy(h_hbm.at[sidx[b]], hm[b], gsem[b])
        pltpu.async_copy(e_hbm.at[pl.ds((base0 + i * CHUNK) // 8, CHUNK // 8)],
                         ev[b], esem[b])

    def wait_in(b):
        pltpu.make_async_copy(h_hbm.at[sidx[b]], hm[b], gsem[b]).wait()
        pltpu.make_async_copy(e_hbm.at[pl.ds(0, CHUNK // 8)],
                              ev[b], esem[b]).wait()

    def compute(b):
        def row(r5, carry):
            for j in range(8):
                r = r5 * 8 + j
                for g in range(H // 16):
                    sl = pl.ds(g * 16, 16)
                    w = ev[b][r5, j, sl]
                    lo = lax.bitcast_convert_type(lax.shift_left(w, 16),
                                                  jnp.float32)
                    hi = lax.bitcast_convert_type(w & jnp.int32(-65536),
                                                  jnp.float32)
                    hm[b][r, sl] = jnp.maximum(hm[b][r, sl] + lo, 0.0)
                    hm[b][r + CHUNK, sl] = jnp.maximum(
                        hm[b][r + CHUNK, sl] + hi, 0.0)
            return carry

        lax.fori_loop(0, CHUNK // 8, row, 0)

    def issue_scatter(b):
        pltpu.async_copy(hm[b], agg_sh.at[didx[b]], ssem[b], add=True)

    def wait_scatter(b):
        pltpu.make_async_copy(hm[b], agg_sh.at[didx[b]], ssem[b]).wait()

    NCH = CHUNKS_PER_W

    def step(i, b, bn, bp):
        @pl.when(i >= 2)
        def _():
            wait_scatter(bn)

        @pl.when(i <= NCH - 2)
        def _():
            issue_didx(i + 1, bn)
            wait_sidx(bn)
            issue_in(i + 1, bn)

        @pl.when(i <= NCH - 3)
        def _():
            issue_sidx(i + 2, bp)

        wait_in(b)
        compute(b)
        wait_didx(b)
        issue_scatter(b)

    issue_sidx(0, 0)
    issue_sidx(1, 1)
    issue_didx(0, 0)
    wait_sidx(0)
    issue_in(0, 0)

    def triple_steps(t, carry):
        i = 3 * t
        step(i, 0, 1, 2)
        step(i + 1, 1, 2, 0)
        step(i + 2, 2, 0, 1)
        return carry

    lax.fori_loop(0, (NCH - 2) // 3, triple_steps, 0)
    step(NCH - 2, 0, 1, 2)
    step(NCH - 1, 1, 2, 0)

    wait_scatter(0)
    wait_scatter(1)
    plsc.subcore_barrier()

    pltpu.sync_copy(agg_sh.at[pl.ds(r0, ROWS_PER_TILE)],
                    out_hbm.at[cid, pl.ds(r0, ROWS_PER_TILE)])

    @pl.when(sid == NUM_TILES - 1)
    def _():
        t0 = ROWS_PER_TILE * NUM_TILES
        pltpu.sync_copy(agg_sh.at[pl.ds(t0, TAIL_ROWS)],
                        out_hbm.at[cid, pl.ds(t0, TAIL_ROWS)])


@functools.cache
def _sc_edge_pass_fn():
    idx = pltpu.VMEM((2 * CHUNK,), jnp.int32)
    buf_e = pltpu.VMEM((CHUNK // 8, 8, H), jnp.int32)
    buf_h = pltpu.VMEM((2 * CHUNK, H), jnp.float32)
    sem = pltpu.SemaphoreType.DMA
    return functools.partial(
        pl.kernel,
        mesh=plsc.VectorSubcoreMesh(core_axis_name="c", subcore_axis_name="s"),
        out_type=jax.ShapeDtypeStruct((NUM_SC, N, H), jnp.float32),
        scratch_types=[
            idx, idx, idx,
            idx, idx, idx,
            buf_h, buf_h, buf_h,
            buf_e, buf_e, buf_e,
            pltpu.VMEM_SHARED((N, H), jnp.float32),
            sem, sem, sem,
            sem, sem, sem,
            sem, sem, sem,
            sem, sem, sem,
            sem, sem, sem,
        ],
    )(_sc_edge_body)


NB = 1000
NGRID = N // NB


def _final_body(parts_ref, wm_ref, bm_ref, batch_ref, wo_ref, bo_ref,
                out_ref, acc_ref):
    i = pl.program_id(0)

    @pl.when(i == 0)
    def _():
        acc_ref[...] = jnp.zeros_like(acc_ref)

    a = parts_ref[0] + parts_ref[1]
    t = jnp.maximum(
        jnp.dot(a, wm_ref[...], preferred_element_type=jnp.float32)
        + bm_ref[...],
        0.0,
    )
    b = batch_ref[0]
    gids = lax.broadcasted_iota(jnp.int32, (G, NB), 0)
    onehot = (b == gids).astype(jnp.float32)
    acc_ref[...] += jnp.dot(onehot, t, preferred_element_type=jnp.float32)

    @pl.when(i == NGRID - 1)
    def _():
        out_ref[...] = (
            jnp.dot(acc_ref[...], wo_ref[...], preferred_element_type=jnp.float32)
            + bo_ref[...]
        )


def _tc_final(parts, W_msg, b_msg, batch3, W_out, b_out):
    return pl.pallas_call(
        _final_body,
        grid=(NGRID,),
        in_specs=[
            pl.BlockSpec((NUM_SC, NB, H), lambda i: (0, i, 0)),
            pl.BlockSpec((H, H), lambda i: (0, 0)),
            pl.BlockSpec((1, H), lambda i: (0, 0)),
            pl.BlockSpec((1, 1, NB), lambda i: (i, 0, 0)),
            pl.BlockSpec((H, OUT), lambda i: (0, 0)),
            pl.BlockSpec((1, OUT), lambda i: (0, 0)),
        ],
        out_specs=pl.BlockSpec((G, OUT), lambda i: (0, 0)),
        out_shape=jax.ShapeDtypeStruct((G, OUT), jnp.float32),
        scratch_shapes=[pltpu.VMEM((G, H), jnp.float32)],
    )(parts, W_msg, b_msg, batch3, W_out, b_out)


def kernel(x, edge_attr, W_node, b_node, W_edge, b_edge, W_msg, b_msg,
           W_out, b_out, edge_index, batch):
    def comb(v):
        shaped = (NW, CHUNKS_PER_W, CHUNK)
        return jnp.concatenate(
            [v[:E2].reshape(shaped), v[E2:].reshape(shaped)], axis=-1
        ).reshape(-1)

    src = comb(edge_index[0])
    dst = comb(edge_index[1])
    W_bd = jnp.kron(jnp.eye(8, dtype=jnp.float32), W_edge)
    b_bd = jnp.tile(b_edge, (8,)).reshape(1, 8 * H)
    e, h = _tc_encode(edge_attr.reshape(E // 8, DF), W_bd, b_bd,
                      x, W_node, b_node.reshape(1, H))
    zeros = jnp.zeros((N, H), jnp.float32)
    parts = _sc_edge_pass_fn()(h, e, src, dst, zeros)
    return _tc_final(parts, W_msg, b_msg.reshape(1, H),
                     batch.reshape(NGRID, 1, NB), W_out, b_out.reshape(1, OUT))


# speedup vs baseline: 1.5565x; 1.5565x over previous
import functools

import jax
import jax.numpy as jnp
from jax import lax
from jax.experimental import pallas as pl
from jax.experimental.pallas import tpu as pltpu
from jax.experimental.pallas import tpu_sc as plsc

N = 10000
E = 320000
E2 = E // 2
DF = 128
DE = 16
H = 128
G = 64
OUT = 64

NUM_SC = 2
NUM_TILES = 16
NW = NUM_SC * NUM_TILES
PAIRS_PER_W = E2 // NW
CHUNK = 40
CHUNKS_PER_W = PAIRS_PER_W // CHUNK
ROWS_PER_TILE = 624
TAIL_ROWS = N - ROWS_PER_TILE * NUM_TILES

EB = 6400
EGRID = E2 // EB


def _encode_body(ea_lo_ref, ea_hi_ref, we_ref, be_ref, x_ref, wn_ref, bn_ref,
                 e_ref, h_ref):
    i = pl.program_id(0)
    web = we_ref[...].astype(jnp.bfloat16)
    dn = (((0,), (0,)), ((), ()))
    u_lo = (
        lax.dot_general(ea_lo_ref[...].astype(jnp.bfloat16), web, dn,
                        preferred_element_type=jnp.float32)
        + be_ref[...]
    )
    u_hi = (
        lax.dot_general(ea_hi_ref[...].astype(jnp.bfloat16), web, dn,
                        preferred_element_type=jnp.float32)
        + be_ref[...]
    )
    bl = lax.bitcast_convert_type(u_lo, jnp.uint32)
    bh = lax.bitcast_convert_type(u_hi, jnp.uint32)
    lo16 = lax.shift_right_logical(bl + jnp.uint32(0x8000), jnp.uint32(16))
    hi16 = (bh + jnp.uint32(0x8000)) & jnp.uint32(0xFFFF0000)
    e_ref[...] = lax.bitcast_convert_type(lo16 | hi16, jnp.int32)

    @pl.when(i == 0)
    def _():
        h_ref[...] = (
            jnp.dot(x_ref[...], wn_ref[...], preferred_element_type=jnp.float32)
            + bn_ref[...]
        )


def _tc_encode(edge_attr, W_edge, b_edge, x, W_node, b_node):
    return pl.pallas_call(
        _encode_body,
        grid=(EGRID,),
        in_specs=[
            pl.BlockSpec((DE, EB), lambda i: (0, i)),
            pl.BlockSpec((DE, EB), lambda i: (0, i + EGRID)),
            pl.BlockSpec((DE, H), lambda i: (0, 0)),
            pl.BlockSpec((1, H), lambda i: (0, 0)),
            pl.BlockSpec((N, DF), lambda i: (0, 0)),
            pl.BlockSpec((DF, H), lambda i: (0, 0)),
            pl.BlockSpec((1, H), lambda i: (0, 0)),
        ],
        out_specs=[
            pl.BlockSpec((EB, H), lambda i: (i, 0)),
            pl.BlockSpec((N, H), lambda i: (0, 0)),
        ],
        out_shape=[
            jax.ShapeDtypeStruct((E2, H), jnp.int32),
            jax.ShapeDtypeStruct((N, H), jnp.float32),
        ],
    )(edge_attr, edge_attr, W_edge, b_edge, x, W_node, b_node)


def _sc_edge_body(h_hbm, e_hbm, src_hbm, dst_hbm, zeros_hbm, out_hbm,
                  si0, si1, si2, di0, di1, di2,
                  hm0, hm1, hm2, ev0, ev1, ev2,
                  agg_sh,
                  xssem0, xssem1, xssem2, xdsem0, xdsem1, xdsem2,
                  gsem0, gsem1, gsem2, esem0, esem1, esem2,
                  ssem0, ssem1, ssem2):
    cid = lax.axis_index("c")
    sid = lax.axis_index("s")
    wid = sid * NUM_SC + cid

    sidx = (si0, si1, si2)
    didx = (di0, di1, di2)
    hm = (hm0, hm1, hm2)
    ev = (ev0, ev1, ev2)
    xssem = (xssem0, xssem1, xssem2)
    xdsem = (xdsem0, xdsem1, xdsem2)
    gsem = (gsem0, gsem1, gsem2)
    esem = (esem0, esem1, esem2)
    ssem = (ssem0, ssem1, ssem2)

    r0 = sid * ROWS_PER_TILE
    pltpu.sync_copy(zeros_hbm.at[pl.ds(r0, ROWS_PER_TILE)],
                    agg_sh.at[pl.ds(r0, ROWS_PER_TILE)])

    @pl.when(sid == NUM_TILES - 1)
    def _():
        t0 = ROWS_PER_TILE * NUM_TILES
        pltpu.sync_copy(zeros_hbm.at[pl.ds(t0, TAIL_ROWS)],
                        agg_sh.at[pl.ds(t0, TAIL_ROWS)])

    base0 = wid * PAIRS_PER_W
    ibase = wid * CHUNKS_PER_W * 2 * CHUNK

    def issue_sidx(i, b):
        pltpu.async_copy(src_hbm.at[pl.ds(ibase + i * 2 * CHUNK, 2 * CHUNK)],
                         sidx[b], xssem[b])

    def wait_sidx(b):
        pltpu.make_async_copy(src_hbm.at[pl.ds(0, 2 * CHUNK)],
                              sidx[b], xssem[b]).wait()

    def issue_didx(i, b):
        pltpu.async_copy(dst_hbm.at[pl.ds(ibase + i * 2 * CHUNK, 2 * CHUNK)],
                         didx[b], xdsem[b])

    def wait_didx(b):
        pltpu.make_async_copy(dst_hbm.at[pl.ds(0, 2 * CHUNK)],
                              didx[b], xdsem[b]).wait()

    def issue_in(i, b):
        pltpu.async_copy(h_hbm.at[sidx[b]], hm[b], gsem[b])
        pltpu.async_copy(e_hbm.at[pl.ds(base0 + i * CHUNK, CHUNK)],
                         ev[b], esem[b])

    def wait_in(b):
        pltpu.make_async_copy(h_hbm.at[sidx[b]], hm[b], gsem[b]).wait()
        pltpu.make_async_copy(e_hbm.at[pl.ds(0, CHUNK)], ev[b], esem[b]).wait()

    def compute(b):
        def row(r, carry):
            for g in range(H // 16):
                sl = pl.ds(g * 16, 16)
                w = ev[b][r, sl]
                lo = lax.bitcast_convert_type(lax.shift_left(w, 16), jnp.float32)
                hi = lax.bitcast_convert_type(w & jnp.int32(-65536), jnp.float32)
                hm[b][r, sl] = jnp.maximum(hm[b][r, sl] + lo, 0.0)
                hm[b][r + CHUNK, sl] = jnp.maximum(hm[b][r + CHUNK, sl] + hi, 0.0)
            return carry

        lax.fori_loop(0, CHUNK, row, 0)

    def issue_scatter(b):
        pltpu.async_copy(hm[b], agg_sh.at[didx[b]], ssem[b], add=True)

    def wait_scatter(b):
        pltpu.make_async_copy(hm[b], agg_sh.at[didx[b]], ssem[b]).wait()

    NCH = CHUNKS_PER_W

    def step(i, b, bn, bp):
        @pl.when(i >= 2)
        def _():
            wait_scatter(bn)

        @pl.when(i <= NCH - 2)
        def _():
            issue_didx(i + 1, bn)
            wait_sidx(bn)
            issue_in(i + 1, bn)

        @pl.when(i <= NCH - 3)
        def _():
            issue_sidx(i + 2, bp)

        wait_in(b)
        compute(b)
        wait_didx(b)
        issue_scatter(b)

    issue_sidx(0, 0)
    issue_sidx(1, 1)
    issue_didx(0, 0)
    wait_sidx(0)
    issue_in(0, 0)

    def triple_steps(t, carry):
        i = 3 * t
        step(i, 0, 1, 2)
        step(i + 1, 1, 2, 0)
        step(i + 2, 2, 0, 1)
        return carry

    lax.fori_loop(0, (NCH - 2) // 3, triple_steps, 0)
    step(NCH - 2, 0, 1, 2)
    step(NCH - 1, 1, 2, 0)

    wait_scatter(0)
    wait_scatter(1)
    plsc.subcore_barrier()

    pltpu.sync_copy(agg_sh.at[pl.ds(r0, ROWS_PER_TILE)],
                    out_hbm.at[cid, pl.ds(r0, ROWS_PER_TILE)])

    @pl.when(sid == NUM_TILES - 1)
    def _():
        t0 = ROWS_PER_TILE * NUM_TILES
        pltpu.sync_copy(agg_sh.at[pl.ds(t0, TAIL_ROWS)],
                        out_hbm.at[cid, pl.ds(t0, TAIL_ROWS)])


@functools.cache
def _sc_edge_pass_fn():
    idx = pltpu.VMEM((2 * CHUNK,), jnp.int32)
    buf_e = pltpu.VMEM((CHUNK, H), jnp.int32)
    buf_h = pltpu.VMEM((2 * CHUNK, H), jnp.float32)
    sem = pltpu.SemaphoreType.DMA
    return functools.partial(
        pl.kernel,
        mesh=plsc.VectorSubcoreMesh(core_axis_name="c", subcore_axis_name="s"),
        out_type=jax.ShapeDtypeStruct((NUM_SC, N, H), jnp.float32),
        scratch_types=[
            idx, idx, idx,
            idx, idx, idx,
            buf_h, buf_h, buf_h,
            buf_e, buf_e, buf_e,
            pltpu.VMEM_SHARED((N, H), jnp.float32),
            sem, sem, sem,
            sem, sem, sem,
            sem, sem, sem,
            sem, sem, sem,
            sem, sem, sem,
        ],
    )(_sc_edge_body)


NB = 1000
NGRID = N // NB


def _final_body(parts_ref, wm_ref, bm_ref, batch_ref, wo_ref, bo_ref,
                out_ref, acc_ref):
    i = pl.program_id(0)

    @pl.when(i == 0)
    def _():
        acc_ref[...] = jnp.zeros_like(acc_ref)

    a = parts_ref[0] + parts_ref[1]
    t = jnp.maximum(
        jnp.dot(a, wm_ref[...], preferred_element_type=jnp.float32)
        + bm_ref[...],
        0.0,
    )
    b = batch_ref[0]
    gids = lax.broadcasted_iota(jnp.int32, (G, NB), 0)
    onehot = (b == gids).astype(jnp.float32)
    acc_ref[...] += jnp.dot(onehot, t, preferred_element_type=jnp.float32)

    @pl.when(i == NGRID - 1)
    def _():
        out_ref[...] = (
            jnp.dot(acc_ref[...], wo_ref[...], preferred_element_type=jnp.float32)
            + bo_ref[...]
        )


def _tc_final(parts, W_msg, b_msg, batch3, W_out, b_out):
    return pl.pallas_call(
        _final_body,
        grid=(NGRID,),
        in_specs=[
            pl.BlockSpec((NUM_SC, NB, H), lambda i: (0, i, 0)),
            pl.BlockSpec((H, H), lambda i: (0, 0)),
            pl.BlockSpec((1, H), lambda i: (0, 0)),
            pl.BlockSpec((1, 1, NB), lambda i: (i, 0, 0)),
            pl.BlockSpec((H, OUT), lambda i: (0, 0)),
            pl.BlockSpec((1, OUT), lambda i: (0, 0)),
        ],
        out_specs=pl.BlockSpec((G, OUT), lambda i: (0, 0)),
        out_shape=jax.ShapeDtypeStruct((G, OUT), jnp.float32),
        scratch_shapes=[pltpu.VMEM((G, H), jnp.float32)],
    )(parts, W_msg, b_msg, batch3, W_out, b_out)


def kernel(x, edge_attr, W_node, b_node, W_edge, b_edge, W_msg, b_msg,
           W_out, b_out, edge_index, batch):
    def comb(v):
        shaped = (NW, CHUNKS_PER_W, CHUNK)
        return jnp.concatenate(
            [v[:E2].reshape(shaped), v[E2:].reshape(shaped)], axis=-1
        ).reshape(-1)

    src = comb(edge_index[0])
    dst = comb(edge_index[1])
    e, h = _tc_encode(edge_attr.T, W_edge, b_edge.reshape(1, H),
                      x, W_node, b_node.reshape(1, H))
    zeros = jnp.zeros((N, H), jnp.float32)
    parts = _sc_edge_pass_fn()(h, e, src, dst, zeros)
    return _tc_final(parts, W_msg, b_msg.reshape(1, H),
                     batch.reshape(NGRID, 1, NB), W_out, b_out.reshape(1, OUT))


# speedup vs baseline: 1.6047x; 1.0309x over previous
import functools

import jax
import jax.numpy as jnp
from jax import lax
from jax.experimental import pallas as pl
from jax.experimental.pallas import tpu as pltpu
from jax.experimental.pallas import tpu_sc as plsc

N = 10000
E = 320000
E2 = E // 2
DF = 128
DE = 16
H = 128
G = 64
OUT = 64

NUM_SC = 2
NUM_TILES = 16
NW = NUM_SC * NUM_TILES
PAIRS_PER_W = E2 // NW
CHUNK = 40
CHUNKS_PER_W = PAIRS_PER_W // CHUNK
ROWS_PER_TILE = 624
TAIL_ROWS = N - ROWS_PER_TILE * NUM_TILES

EB = 32000
EGRID = E2 // EB


def _encode_body(ea_lo_ref, ea_hi_ref, we_ref, be_ref, x_ref, wn_ref, bn_ref,
                 e_ref, h_ref):
    i = pl.program_id(0)
    web = we_ref[...].astype(jnp.bfloat16)
    dn = (((0,), (0,)), ((), ()))
    u_lo = (
        lax.dot_general(ea_lo_ref[...].astype(jnp.bfloat16), web, dn,
                        preferred_element_type=jnp.float32)
        + be_ref[...]
    )
    u_hi = (
        lax.dot_general(ea_hi_ref[...].astype(jnp.bfloat16), web, dn,
                        preferred_element_type=jnp.float32)
        + be_ref[...]
    )
    bl = lax.bitcast_convert_type(u_lo, jnp.uint32)
    bh = lax.bitcast_convert_type(u_hi, jnp.uint32)
    lo16 = lax.shift_right_logical(bl + jnp.uint32(0x8000), jnp.uint32(16))
    hi16 = (bh + jnp.uint32(0x8000)) & jnp.uint32(0xFFFF0000)
    e_ref[...] = lax.bitcast_convert_type(lo16 | hi16, jnp.int32)

    @pl.when(i == 0)
    def _():
        h_ref[...] = (
            jnp.dot(x_ref[...], wn_ref[...], preferred_element_type=jnp.float32)
            + bn_ref[...]
        )


def _tc_encode(edge_attr, W_edge, b_edge, x, W_node, b_node):
    return pl.pallas_call(
        _encode_body,
        grid=(EGRID,),
        in_specs=[
            pl.BlockSpec((DE, EB), lambda i: (0, i)),
            pl.BlockSpec((DE, EB), lambda i: (0, i + EGRID)),
            pl.BlockSpec((DE, H), lambda i: (0, 0)),
            pl.BlockSpec((1, H), lambda i: (0, 0)),
            pl.BlockSpec((N, DF), lambda i: (0, 0)),
            pl.BlockSpec((DF, H), lambda i: (0, 0)),
            pl.BlockSpec((1, H), lambda i: (0, 0)),
        ],
        out_specs=[
            pl.BlockSpec((EB, H), lambda i: (i, 0)),
            pl.BlockSpec((N, H), lambda i: (0, 0)),
        ],
        out_shape=[
            jax.ShapeDtypeStruct((E2, H), jnp.int32),
            jax.ShapeDtypeStruct((N, H), jnp.float32),
        ],
    )(edge_attr, edge_attr, W_edge, b_edge, x, W_node, b_node)


def _sc_edge_body(h_hbm, e_hbm, src_hbm, dst_hbm, zeros_hbm, out_hbm,
                  si0, si1, si2, di0, di1, di2,
                  hm0, hm1, hm2, ev0, ev1, ev2,
                  agg_sh,
                  xssem0, xssem1, xssem2, xdsem0, xdsem1, xdsem2,
                  gsem0, gsem1, gsem2, esem0, esem1, esem2,
                  ssem0, ssem1, ssem2):
    cid = lax.axis_index("c")
    sid = lax.axis_index("s")
    wid = sid * NUM_SC + cid

    sidx = (si0, si1, si2)
    didx = (di0, di1, di2)
    hm = (hm0, hm1, hm2)
    ev = (ev0, ev1, ev2)
    xssem = (xssem0, xssem1, xssem2)
    xdsem = (xdsem0, xdsem1, xdsem2)
    gsem = (gsem0, gsem1, gsem2)
    esem = (esem0, esem1, esem2)
    ssem = (ssem0, ssem1, ssem2)

    r0 = sid * ROWS_PER_TILE
    pltpu.sync_copy(zeros_hbm.at[pl.ds(r0, ROWS_PER_TILE)],
                    agg_sh.at[pl.ds(r0, ROWS_PER_TILE)])

    @pl.when(sid == NUM_TILES - 1)
    def _():
        t0 = ROWS_PER_TILE * NUM_TILES
        pltpu.sync_copy(zeros_hbm.at[pl.ds(t0, TAIL_ROWS)],
                        agg_sh.at[pl.ds(t0, TAIL_ROWS)])

    base0 = wid * PAIRS_PER_W
    ibase = wid * CHUNKS_PER_W * 2 * CHUNK

    def issue_sidx(i, b):
        pltpu.async_copy(src_hbm.at[pl.ds(ibase + i * 2 * CHUNK, 2 * CHUNK)],
                         sidx[b], xssem[b])

    def wait_sidx(b):
        pltpu.make_async_copy(src_hbm.at[pl.ds(0, 2 * CHUNK)],
                              sidx[b], xssem[b]).wait()

    def issue_didx(i, b):
        pltpu.async_copy(dst_hbm.at[pl.ds(ibase + i * 2 * CHUNK, 2 * CHUNK)],
                         didx[b], xdsem[b])

    def wait_didx(b):
        pltpu.make_async_copy(dst_hbm.at[pl.ds(0, 2 * CHUNK)],
                              didx[b], xdsem[b]).wait()

    def issue_in(i, b):
        pltpu.async_copy(h_hbm.at[sidx[b]], hm[b], gsem[b])
        pltpu.async_copy(e_hbm.at[pl.ds(base0 + i * CHUNK, CHUNK)],
                         ev[b], esem[b])

    def wait_in(b):
        pltpu.make_async_copy(h_hbm.at[sidx[b]], hm[b], gsem[b]).wait()
        pltpu.make_async_copy(e_hbm.at[pl.ds(0, CHUNK)], ev[b], esem[b]).wait()

    def compute(b):
        def row(r, carry):
            for g in range(H // 16):
                sl = pl.ds(g * 16, 16)
                w = ev[b][r, sl]
                lo = lax.bitcast_convert_type(lax.shift_left(w, 16), jnp.float32)
                hi = lax.bitcast_convert_type(w & jnp.int32(-65536), jnp.float32)
                hm[b][r, sl] = jnp.maximum(hm[b][r, sl] + lo, 0.0)
                hm[b][r + CHUNK, sl] = jnp.maximum(hm[b][r + CHUNK, sl] + hi, 0.0)
            return carry

        lax.fori_loop(0, CHUNK, row, 0)

    def issue_scatter(b):
        pltpu.async_copy(hm[b], agg_sh.at[didx[b]], ssem[b], add=True)

    def wait_scatter(b):
        pltpu.make_async_copy(hm[b], agg_sh.at[didx[b]], ssem[b]).wait()

    NCH = CHUNKS_PER_W

    def step(i, b, bn, bp):
        @pl.when(i >= 2)
        def _():
            wait_scatter(bn)

        @pl.when(i <= NCH - 2)
        def _():
            issue_didx(i + 1, bn)
            wait_sidx(bn)
            issue_in(i + 1, bn)

        @pl.when(i <= NCH - 3)
        def _():
            issue_sidx(i + 2, bp)

        wait_in(b)
        compute(b)
        wait_didx(b)
        issue_scatter(b)

    issue_sidx(0, 0)
    issue_sidx(1, 1)
    issue_didx(0, 0)
    wait_sidx(0)
    issue_in(0, 0)

    def triple_steps(t, carry):
        i = 3 * t
        step(i, 0, 1, 2)
        step(i + 1, 1, 2, 0)
        step(i + 2, 2, 0, 1)
        return carry

    lax.fori_loop(0, (NCH - 2) // 3, triple_steps, 0)
    step(NCH - 2, 0, 1, 2)
    step(NCH - 1, 1, 2, 0)

    wait_scatter(0)
    wait_scatter(1)
    plsc.subcore_barrier()

    pltpu.sync_copy(agg_sh.at[pl.ds(r0, ROWS_PER_TILE)],
                    out_hbm.at[cid, pl.ds(r0, ROWS_PER_TILE)])

    @pl.when(sid == NUM_TILES - 1)
    def _():
        t0 = ROWS_PER_TILE * NUM_TILES
        pltpu.sync_copy(agg_sh.at[pl.ds(t0, TAIL_ROWS)],
                        out_hbm.at[cid, pl.ds(t0, TAIL_ROWS)])


@functools.cache
def _sc_edge_pass_fn():
    idx = pltpu.VMEM((2 * CHUNK,), jnp.int32)
    buf_e = pltpu.VMEM((CHUNK, H), jnp.int32)
    buf_h = pltpu.VMEM((2 * CHUNK, H), jnp.float32)
    sem = pltpu.SemaphoreType.DMA
    return functools.partial(
        pl.kernel,
        mesh=plsc.VectorSubcoreMesh(core_axis_name="c", subcore_axis_name="s"),
        out_type=jax.ShapeDtypeStruct((NUM_SC, N, H), jnp.float32),
        scratch_types=[
            idx, idx, idx,
            idx, idx, idx,
            buf_h, buf_h, buf_h,
            buf_e, buf_e, buf_e,
            pltpu.VMEM_SHARED((N, H), jnp.float32),
            sem, sem, sem,
            sem, sem, sem,
            sem, sem, sem,
            sem, sem, sem,
            sem, sem, sem,
        ],
    )(_sc_edge_body)


NB = 1000
NGRID = N // NB


def _final_body(parts_ref, wm_ref, bm_ref, batch_ref, wo_ref, bo_ref,
                out_ref, acc_ref):
    i = pl.program_id(0)

    @pl.when(i == 0)
    def _():
        acc_ref[...] = jnp.zeros_like(acc_ref)

    a = parts_ref[0] + parts_ref[1]
    t = jnp.maximum(
        jnp.dot(a, wm_ref[...], preferred_element_type=jnp.float32)
        + bm_ref[...],
        0.0,
    )
    b = batch_ref[0]
    gids = lax.broadcasted_iota(jnp.int32, (G, NB), 0)
    onehot = (b == gids).astype(jnp.float32)
    acc_ref[...] += jnp.dot(onehot, t, preferred_element_type=jnp.float32)

    @pl.when(i == NGRID - 1)
    def _():
        out_ref[...] = (
            jnp.dot(acc_ref[...], wo_ref[...], preferred_element_type=jnp.float32)
            + bo_ref[...]
        )


def _tc_final(parts, W_msg, b_msg, batch3, W_out, b_out):
    return pl.pallas_call(
        _final_body,
        grid=(NGRID,),
        in_specs=[
            pl.BlockSpec((NUM_SC, NB, H), lambda i: (0, i, 0)),
            pl.BlockSpec((H, H), lambda i: (0, 0)),
            pl.BlockSpec((1, H), lambda i: (0, 0)),
            pl.BlockSpec((1, 1, NB), lambda i: (i, 0, 0)),
            pl.BlockSpec((H, OUT), lambda i: (0, 0)),
            pl.BlockSpec((1, OUT), lambda i: (0, 0)),
        ],
        out_specs=pl.BlockSpec((G, OUT), lambda i: (0, 0)),
        out_shape=jax.ShapeDtypeStruct((G, OUT), jnp.float32),
        scratch_shapes=[pltpu.VMEM((G, H), jnp.float32)],
    )(parts, W_msg, b_msg, batch3, W_out, b_out)


def kernel(x, edge_attr, W_node, b_node, W_edge, b_edge, W_msg, b_msg,
           W_out, b_out, edge_index, batch):
    def comb(v):
        shaped = (NW, CHUNKS_PER_W, CHUNK)
        return jnp.concatenate(
            [v[:E2].reshape(shaped), v[E2:].reshape(shaped)], axis=-1
        ).reshape(-1)

    src = comb(edge_index[0])
    dst = comb(edge_index[1])
    e, h = _tc_encode(edge_attr.T, W_edge, b_edge.reshape(1, H),
                      x, W_node, b_node.reshape(1, H))
    zeros = jnp.zeros((N, H), jnp.float32)
    parts = _sc_edge_pass_fn()(h, e, src, dst, zeros)
    return _tc_final(parts, W_msg, b_msg.reshape(1, H),
                     batch.reshape(NGRID, 1, NB), W_out, b_out.reshape(1, OUT))


# speedup vs baseline: 1.8110x; 1.1286x over previous
import functools

import jax
import jax.numpy as jnp
from jax import lax
from jax.experimental import pallas as pl
from jax.experimental.pallas import tpu as pltpu
from jax.experimental.pallas import tpu_sc as plsc

N = 10000
E = 320000
E2 = E // 2
DF = 128
DE = 16
H = 128
G = 64
OUT = 64

NUM_SC = 2
NUM_TILES = 16
NW = NUM_SC * NUM_TILES
PAIRS_PER_W = E2 // NW
CHUNK = 40
CHUNKS_PER_W = PAIRS_PER_W // CHUNK
ROWS_PER_TILE = 624
TAIL_ROWS = N - ROWS_PER_TILE * NUM_TILES

EB = 32000
EGRID = E2 // EB


def _encode_body(ea_lo_ref, ea_hi_ref, we_ref, be_ref, x_ref, wn_ref, bn_ref,
                 e_ref, h_ref):
    i = pl.program_id(0)
    web = we_ref[...].astype(jnp.bfloat16)
    dn = (((0,), (0,)), ((), ()))
    u_lo = (
        lax.dot_general(ea_lo_ref[...].astype(jnp.bfloat16), web, dn,
                        preferred_element_type=jnp.float32)
        + be_ref[...]
    )
    u_hi = (
        lax.dot_general(ea_hi_ref[...].astype(jnp.bfloat16), web, dn,
                        preferred_element_type=jnp.float32)
        + be_ref[...]
    )
    bl = lax.bitcast_convert_type(u_lo, jnp.uint32)
    bh = lax.bitcast_convert_type(u_hi, jnp.uint32)
    lo16 = lax.shift_right_logical(bl + jnp.uint32(0x8000), jnp.uint32(16))
    hi16 = (bh + jnp.uint32(0x8000)) & jnp.uint32(0xFFFF0000)
    e_ref[...] = lax.bitcast_convert_type(lo16 | hi16, jnp.int32)

    @pl.when(i == 0)
    def _():
        h_ref[...] = (
            jnp.dot(x_ref[...], wn_ref[...], preferred_element_type=jnp.float32)
            + bn_ref[...]
        )


def _tc_encode(edge_attr, W_edge, b_edge, x, W_node, b_node):
    return pl.pallas_call(
        _encode_body,
        grid=(EGRID,),
        in_specs=[
            pl.BlockSpec((DE, EB), lambda i: (0, i)),
            pl.BlockSpec((DE, EB), lambda i: (0, i + EGRID)),
            pl.BlockSpec((DE, H), lambda i: (0, 0)),
            pl.BlockSpec((1, H), lambda i: (0, 0)),
            pl.BlockSpec((N, DF), lambda i: (0, 0)),
            pl.BlockSpec((DF, H), lambda i: (0, 0)),
            pl.BlockSpec((1, H), lambda i: (0, 0)),
        ],
        out_specs=[
            pl.BlockSpec((EB, H), lambda i: (i, 0)),
            pl.BlockSpec((N, H), lambda i: (0, 0)),
        ],
        out_shape=[
            jax.ShapeDtypeStruct((E2, H), jnp.int32),
            jax.ShapeDtypeStruct((N, H), jnp.float32),
        ],
    )(edge_attr, edge_attr, W_edge, b_edge, x, W_node, b_node)


def _sc_edge_body(h_hbm, e_hbm, src_hbm, dst_hbm, zeros_hbm, out_hbm,
                  si0, si1, si2, di0, di1, di2,
                  hm0, hm1, hm2, ev0, ev1, ev2,
                  agg_sh,
                  xssem0, xssem1, xssem2, xdsem0, xdsem1, xdsem2,
                  gsem0, gsem1, gsem2, esem0, esem1, esem2,
                  ssem0, ssem1, ssem2):
    cid = lax.axis_index("c")
    sid = lax.axis_index("s")
    wid = sid * NUM_SC + cid

    sidx = (si0, si1, si2)
    didx = (di0, di1, di2)
    hm = (hm0, hm1, hm2)
    ev = (ev0, ev1, ev2)
    xssem = (xssem0, xssem1, xssem2)
    xdsem = (xdsem0, xdsem1, xdsem2)
    gsem = (gsem0, gsem1, gsem2)
    esem = (esem0, esem1, esem2)
    ssem = (ssem0, ssem1, ssem2)

    r0 = sid * ROWS_PER_TILE
    pltpu.sync_copy(zeros_hbm.at[pl.ds(r0, ROWS_PER_TILE)],
                    agg_sh.at[pl.ds(r0, ROWS_PER_TILE)])

    @pl.when(sid == NUM_TILES - 1)
    def _():
        t0 = ROWS_PER_TILE * NUM_TILES
        pltpu.sync_copy(zeros_hbm.at[pl.ds(t0, TAIL_ROWS)],
                        agg_sh.at[pl.ds(t0, TAIL_ROWS)])

    base0 = wid * PAIRS_PER_W
    ibase = wid * CHUNKS_PER_W * 2 * CHUNK

    def issue_sidx(i, b):
        lo = base0 + i * CHUNK
        pltpu.async_copy(src_hbm.at[pl.ds(lo, CHUNK)],
                         sidx[b].at[pl.ds(0, CHUNK)], xssem[b])
        pltpu.async_copy(src_hbm.at[pl.ds(E2 + lo, CHUNK)],
                         sidx[b].at[pl.ds(CHUNK, CHUNK)], xssem[b])

    def wait_sidx(b):
        pltpu.make_async_copy(src_hbm.at[pl.ds(0, CHUNK)],
                              sidx[b].at[pl.ds(0, CHUNK)], xssem[b]).wait()
        pltpu.make_async_copy(src_hbm.at[pl.ds(0, CHUNK)],
                              sidx[b].at[pl.ds(CHUNK, CHUNK)], xssem[b]).wait()

    def issue_didx(i, b):
        lo = base0 + i * CHUNK
        pltpu.async_copy(dst_hbm.at[pl.ds(lo, CHUNK)],
                         didx[b].at[pl.ds(0, CHUNK)], xdsem[b])
        pltpu.async_copy(dst_hbm.at[pl.ds(E2 + lo, CHUNK)],
                         didx[b].at[pl.ds(CHUNK, CHUNK)], xdsem[b])

    def wait_didx(b):
        pltpu.make_async_copy(dst_hbm.at[pl.ds(0, CHUNK)],
                              didx[b].at[pl.ds(0, CHUNK)], xdsem[b]).wait()
        pltpu.make_async_copy(dst_hbm.at[pl.ds(0, CHUNK)],
                              didx[b].at[pl.ds(CHUNK, CHUNK)], xdsem[b]).wait()

    def issue_in(i, b):
        pltpu.async_copy(h_hbm.at[sidx[b]], hm[b], gsem[b])
        pltpu.async_copy(e_hbm.at[pl.ds(base0 + i * CHUNK, CHUNK)],
                         ev[b], esem[b])

    def wait_in(b):
        pltpu.make_async_copy(h_hbm.at[sidx[b]], hm[b], gsem[b]).wait()
        pltpu.make_async_copy(e_hbm.at[pl.ds(0, CHUNK)], ev[b], esem[b]).wait()

    def compute(b):
        def row(r, carry):
            for g in range(H // 16):
                sl = pl.ds(g * 16, 16)
                w = ev[b][r, sl]
                lo = lax.bitcast_convert_type(lax.shift_left(w, 16), jnp.float32)
                hi = lax.bitcast_convert_type(w & jnp.int32(-65536), jnp.float32)
                hm[b][r, sl] = jnp.maximum(hm[b][r, sl] + lo, 0.0)
                hm[b][r + CHUNK, sl] = jnp.maximum(hm[b][r + CHUNK, sl] + hi, 0.0)
            return carry

        lax.fori_loop(0, CHUNK, row, 0)

    def issue_scatter(b):
        pltpu.async_copy(hm[b], agg_sh.at[didx[b]], ssem[b], add=True)

    def wait_scatter(b):
        pltpu.make_async_copy(hm[b], agg_sh.at[didx[b]], ssem[b]).wait()

    NCH = CHUNKS_PER_W

    def step(i, b, bn, bp):
        @pl.when(i >= 2)
        def _():
            wait_scatter(bn)

        @pl.when(i <= NCH - 2)
        def _():
            issue_didx(i + 1, bn)
            wait_sidx(bn)
            issue_in(i + 1, bn)

        @pl.when(i <= NCH - 3)
        def _():
            issue_sidx(i + 2, bp)

        wait_in(b)
        compute(b)
        wait_didx(b)
        issue_scatter(b)

    issue_sidx(0, 0)
    issue_sidx(1, 1)
    issue_didx(0, 0)
    wait_sidx(0)
    issue_in(0, 0)

    def triple_steps(t, carry):
        i = 3 * t
        step(i, 0, 1, 2)
        step(i + 1, 1, 2, 0)
        step(i + 2, 2, 0, 1)
        return carry

    lax.fori_loop(0, (NCH - 2) // 3, triple_steps, 0)
    step(NCH - 2, 0, 1, 2)
    step(NCH - 1, 1, 2, 0)

    wait_scatter(0)
    wait_scatter(1)
    plsc.subcore_barrier()

    pltpu.sync_copy(agg_sh.at[pl.ds(r0, ROWS_PER_TILE)],
                    out_hbm.at[cid, pl.ds(r0, ROWS_PER_TILE)])

    @pl.when(sid == NUM_TILES - 1)
    def _():
        t0 = ROWS_PER_TILE * NUM_TILES
        pltpu.sync_copy(agg_sh.at[pl.ds(t0, TAIL_ROWS)],
                        out_hbm.at[cid, pl.ds(t0, TAIL_ROWS)])


@functools.cache
def _sc_edge_pass_fn():
    idx = pltpu.VMEM((2 * CHUNK,), jnp.int32)
    buf_e = pltpu.VMEM((CHUNK, H), jnp.int32)
    buf_h = pltpu.VMEM((2 * CHUNK, H), jnp.float32)
    sem = pltpu.SemaphoreType.DMA
    return functools.partial(
        pl.kernel,
        mesh=plsc.VectorSubcoreMesh(core_axis_name="c", subcore_axis_name="s"),
        out_type=jax.ShapeDtypeStruct((NUM_SC, N, H), jnp.float32),
        scratch_types=[
            idx, idx, idx,
            idx, idx, idx,
            buf_h, buf_h, buf_h,
            buf_e, buf_e, buf_e,
            pltpu.VMEM_SHARED((N, H), jnp.float32),
            sem, sem, sem,
            sem, sem, sem,
            sem, sem, sem,
            sem, sem, sem,
            sem, sem, sem,
        ],
    )(_sc_edge_body)


NB = 1000
NGRID = N // NB


def _final_body(parts_ref, wm_ref, bm_ref, batch_ref, wo_ref, bo_ref,
                out_ref, acc_ref):
    i = pl.program_id(0)

    @pl.when(i == 0)
    def _():
        acc_ref[...] = jnp.zeros_like(acc_ref)

    a = parts_ref[0] + parts_ref[1]
    t = jnp.maximum(
        jnp.dot(a, wm_ref[...], preferred_element_type=jnp.float32)
        + bm_ref[...],
        0.0,
    )
    b = batch_ref[0]
    gids = lax.broadcasted_iota(jnp.int32, (G, NB), 0)
    onehot = (b == gids).astype(jnp.float32)
    acc_ref[...] += jnp.dot(onehot, t, preferred_element_type=jnp.float32)

    @pl.when(i == NGRID - 1)
    def _():
        out_ref[...] = (
            jnp.dot(acc_ref[...], wo_ref[...], preferred_element_type=jnp.float32)
            + bo_ref[...]
        )


def _tc_final(parts, W_msg, b_msg, batch3, W_out, b_out):
    return pl.pallas_call(
        _final_body,
        grid=(NGRID,),
        in_specs=[
            pl.BlockSpec((NUM_SC, NB, H), lambda i: (0, i, 0)),
            pl.BlockSpec((H, H), lambda i: (0, 0)),
            pl.BlockSpec((1, H), lambda i: (0, 0)),
            pl.BlockSpec((1, 1, NB), lambda i: (i, 0, 0)),
            pl.BlockSpec((H, OUT), lambda i: (0, 0)),
            pl.BlockSpec((1, OUT), lambda i: (0, 0)),
        ],
        out_specs=pl.BlockSpec((G, OUT), lambda i: (0, 0)),
        out_shape=jax.ShapeDtypeStruct((G, OUT), jnp.float32),
        scratch_shapes=[pltpu.VMEM((G, H), jnp.float32)],
    )(parts, W_msg, b_msg, batch3, W_out, b_out)


def kernel(x, edge_attr, W_node, b_node, W_edge, b_edge, W_msg, b_msg,
           W_out, b_out, edge_index, batch):
    src = edge_index[0]
    dst = edge_index[1]
    e, h = _tc_encode(edge_attr.T, W_edge, b_edge.reshape(1, H),
                      x, W_node, b_node.reshape(1, H))
    zeros = jnp.zeros((N, H), jnp.float32)
    parts = _sc_edge_pass_fn()(h, e, src, dst, zeros)
    return _tc_final(parts, W_msg, b_msg.reshape(1, H),
                     batch.reshape(NGRID, 1, NB), W_out, b_out.reshape(1, OUT))


# speedup vs baseline: 1.8131x; 1.0011x over previous
import functools

import jax
import jax.numpy as jnp
from jax import lax
from jax.experimental import pallas as pl
from jax.experimental.pallas import tpu as pltpu
from jax.experimental.pallas import tpu_sc as plsc

N = 10000
E = 320000
E2 = E // 2
DF = 128
DE = 16
H = 128
G = 64
OUT = 64

NUM_SC = 2
NUM_TILES = 16
NW = NUM_SC * NUM_TILES
PAIRS_PER_W = E2 // NW
CHUNK = 40
CHUNKS_PER_W = PAIRS_PER_W // CHUNK
ROWS_PER_TILE = 624
TAIL_ROWS = N - ROWS_PER_TILE * NUM_TILES

EB = 32000
EGRID = E2 // EB


def _encode_body(ea_lo_ref, ea_hi_ref, we_ref, be_ref, x_ref, wn_ref, bn_ref,
                 e_ref, h_ref):
    i = pl.program_id(0)
    web = we_ref[...].astype(jnp.bfloat16)
    dn = (((0,), (0,)), ((), ()))
    u_lo = (
        lax.dot_general(ea_lo_ref[...].astype(jnp.bfloat16), web, dn,
                        preferred_element_type=jnp.float32)
        + be_ref[...]
    )
    u_hi = (
        lax.dot_general(ea_hi_ref[...].astype(jnp.bfloat16), web, dn,
                        preferred_element_type=jnp.float32)
        + be_ref[...]
    )
    bl = lax.bitcast_convert_type(u_lo, jnp.uint32)
    bh = lax.bitcast_convert_type(u_hi, jnp.uint32)
    lo16 = lax.shift_right_logical(bl + jnp.uint32(0x8000), jnp.uint32(16))
    hi16 = (bh + jnp.uint32(0x8000)) & jnp.uint32(0xFFFF0000)
    e_ref[...] = lax.bitcast_convert_type(lo16 | hi16, jnp.int32)

    @pl.when(i == 0)
    def _():
        h_ref[...] = (
            jnp.dot(x_ref[...], wn_ref[...], preferred_element_type=jnp.float32)
            + bn_ref[...]
        )


def _tc_encode(edge_attr, W_edge, b_edge, x, W_node, b_node):
    return pl.pallas_call(
        _encode_body,
        grid=(EGRID,),
        in_specs=[
            pl.BlockSpec((DE, EB), lambda i: (0, i)),
            pl.BlockSpec((DE, EB), lambda i: (0, i + EGRID)),
            pl.BlockSpec((DE, H), lambda i: (0, 0)),
            pl.BlockSpec((1, H), lambda i: (0, 0)),
            pl.BlockSpec((N, DF), lambda i: (0, 0)),
            pl.BlockSpec((DF, H), lambda i: (0, 0)),
            pl.BlockSpec((1, H), lambda i: (0, 0)),
        ],
        out_specs=[
            pl.BlockSpec((EB, H), lambda i: (i, 0)),
            pl.BlockSpec((N, H), lambda i: (0, 0)),
        ],
        out_shape=[
            jax.ShapeDtypeStruct((E2, H), jnp.int32),
            jax.ShapeDtypeStruct((N, H), jnp.float32),
        ],
    )(edge_attr, edge_attr, W_edge, b_edge, x, W_node, b_node)


def _sc_edge_body(h_hbm, e_hbm, src_hbm, dst_hbm, zeros_hbm, out_hbm,
                  si0, si1, si2, di0, di1, di2,
                  hm0, hm1, hm2, ev0, ev1, ev2,
                  agg_sh,
                  xssem0, xssem1, xssem2, xdsem0, xdsem1, xdsem2,
                  gsem0, gsem1, gsem2, esem0, esem1, esem2,
                  ssem0, ssem1, ssem2):
    cid = lax.axis_index("c")
    sid = lax.axis_index("s")
    wid = sid * NUM_SC + cid

    sidx = (si0, si1, si2)
    didx = (di0, di1, di2)
    hm = (hm0, hm1, hm2)
    ev = (ev0, ev1, ev2)
    xssem = (xssem0, xssem1, xssem2)
    xdsem = (xdsem0, xdsem1, xdsem2)
    gsem = (gsem0, gsem1, gsem2)
    esem = (esem0, esem1, esem2)
    ssem = (ssem0, ssem1, ssem2)

    r0 = sid * ROWS_PER_TILE
    pltpu.sync_copy(zeros_hbm.at[pl.ds(r0, ROWS_PER_TILE)],
                    agg_sh.at[pl.ds(r0, ROWS_PER_TILE)])

    @pl.when(sid == NUM_TILES - 1)
    def _():
        t0 = ROWS_PER_TILE * NUM_TILES
        pltpu.sync_copy(zeros_hbm.at[pl.ds(t0, TAIL_ROWS)],
                        agg_sh.at[pl.ds(t0, TAIL_ROWS)])

    base0 = wid * PAIRS_PER_W

    def issue_sidx(i, b):
        lo = base0 + i * CHUNK
        pltpu.async_copy(src_hbm.at[pl.ds(lo, CHUNK)],
                         sidx[b].at[pl.ds(0, CHUNK)], xssem[b])
        pltpu.async_copy(src_hbm.at[pl.ds(E2 + lo, CHUNK)],
                         sidx[b].at[pl.ds(CHUNK, CHUNK)], xssem[b])

    def wait_sidx(b):
        pltpu.make_async_copy(src_hbm.at[pl.ds(0, CHUNK)],
                              sidx[b].at[pl.ds(0, CHUNK)], xssem[b]).wait()
        pltpu.make_async_copy(src_hbm.at[pl.ds(0, CHUNK)],
                              sidx[b].at[pl.ds(CHUNK, CHUNK)], xssem[b]).wait()

    def issue_didx(i, b):
        lo = base0 + i * CHUNK
        pltpu.async_copy(dst_hbm.at[pl.ds(lo, CHUNK)],
                         didx[b].at[pl.ds(0, CHUNK)], xdsem[b])
        pltpu.async_copy(dst_hbm.at[pl.ds(E2 + lo, CHUNK)],
                         didx[b].at[pl.ds(CHUNK, CHUNK)], xdsem[b])

    def wait_didx(b):
        pltpu.make_async_copy(dst_hbm.at[pl.ds(0, CHUNK)],
                              didx[b].at[pl.ds(0, CHUNK)], xdsem[b]).wait()
        pltpu.make_async_copy(dst_hbm.at[pl.ds(0, CHUNK)],
                              didx[b].at[pl.ds(CHUNK, CHUNK)], xdsem[b]).wait()

    def issue_in(i, b):
        pltpu.async_copy(h_hbm.at[sidx[b]], hm[b], gsem[b])
        pltpu.async_copy(e_hbm.at[pl.ds(base0 + i * CHUNK, CHUNK)],
                         ev[b], esem[b])

    def wait_in(b):
        pltpu.make_async_copy(h_hbm.at[sidx[b]], hm[b], gsem[b]).wait()
        pltpu.make_async_copy(e_hbm.at[pl.ds(0, CHUNK)], ev[b], esem[b]).wait()

    def compute(b):
        def row(r, carry):
            for g in range(H // 16):
                sl = pl.ds(g * 16, 16)
                w = ev[b][r, sl]
                lo = lax.bitcast_convert_type(lax.shift_left(w, 16), jnp.float32)
                hi = lax.bitcast_convert_type(w & jnp.int32(-65536), jnp.float32)
                hm[b][r, sl] = jnp.maximum(hm[b][r, sl] + lo, 0.0)
                hm[b][r + CHUNK, sl] = jnp.maximum(hm[b][r + CHUNK, sl] + hi, 0.0)
            return carry

        lax.fori_loop(0, CHUNK, row, 0)

    def issue_scatter(b):
        pltpu.async_copy(hm[b], agg_sh.at[didx[b]], ssem[b], add=True)

    def wait_scatter(b):
        pltpu.make_async_copy(hm[b], agg_sh.at[didx[b]], ssem[b]).wait()

    NCH = CHUNKS_PER_W

    def step(i, b, bn, bp):
        @pl.when(i >= 2)
        def _():
            wait_scatter(bn)

        @pl.when(i <= NCH - 2)
        def _():
            issue_didx(i + 1, bn)
            wait_sidx(bn)
            issue_in(i + 1, bn)

        @pl.when(i <= NCH - 3)
        def _():
            issue_sidx(i + 2, bp)

        wait_in(b)
        compute(b)
        wait_didx(b)
        issue_scatter(b)

    issue_sidx(0, 0)
    issue_sidx(1, 1)
    issue_didx(0, 0)
    wait_sidx(0)
    issue_in(0, 0)

    def triple_steps(t, carry):
        i = 3 * t
        step(i, 0, 1, 2)
        step(i + 1, 1, 2, 0)
        step(i + 2, 2, 0, 1)
        return carry

    lax.fori_loop(0, (NCH - 2) // 3, triple_steps, 0)
    step(NCH - 2, 0, 1, 2)
    step(NCH - 1, 1, 2, 0)

    wait_scatter(0)
    wait_scatter(1)
    plsc.subcore_barrier()

    pltpu.sync_copy(agg_sh.at[pl.ds(r0, ROWS_PER_TILE)],
                    out_hbm.at[cid, pl.ds(r0, ROWS_PER_TILE)])

    @pl.when(sid == NUM_TILES - 1)
    def _():
        t0 = ROWS_PER_TILE * NUM_TILES
        pltpu.sync_copy(agg_sh.at[pl.ds(t0, TAIL_ROWS)],
                        out_hbm.at[cid, pl.ds(t0, TAIL_ROWS)])


@functools.cache
def _sc_edge_pass_fn():
    idx = pltpu.VMEM((2 * CHUNK,), jnp.int32)
    buf_e = pltpu.VMEM((CHUNK, H), jnp.int32)
    buf_h = pltpu.VMEM((2 * CHUNK, H), jnp.float32)
    sem = pltpu.SemaphoreType.DMA
    return functools.partial(
        pl.kernel,
        mesh=plsc.VectorSubcoreMesh(core_axis_name="c", subcore_axis_name="s"),
        out_type=jax.ShapeDtypeStruct((NUM_SC, N, H), jnp.float32),
        scratch_types=[
            idx, idx, idx,
            idx, idx, idx,
            buf_h, buf_h, buf_h,
            buf_e, buf_e, buf_e,
            pltpu.VMEM_SHARED((N, H), jnp.float32),
            sem, sem, sem,
            sem, sem, sem,
            sem, sem, sem,
            sem, sem, sem,
            sem, sem, sem,
        ],
    )(_sc_edge_body)


NB = 1000
NGRID = N // NB


def _final_body(parts_ref, wm_ref, bm_ref, batch_ref, wo_ref, bo_ref,
                out_ref, acc_ref):
    i = pl.program_id(0)

    @pl.when(i == 0)
    def _():
        acc_ref[...] = jnp.zeros_like(acc_ref)

    a = parts_ref[0] + parts_ref[1]
    t = jnp.maximum(
        jnp.dot(a, wm_ref[...], preferred_element_type=jnp.float32)
        + bm_ref[...],
        0.0,
    )
    b = batch_ref[0]
    gids = lax.broadcasted_iota(jnp.int32, (G, NB), 0)
    onehot = (b == gids).astype(jnp.float32)
    acc_ref[...] += jnp.dot(onehot, t, preferred_element_type=jnp.float32)

    @pl.when(i == NGRID - 1)
    def _():
        out_ref[...] = (
            jnp.dot(acc_ref[...], wo_ref[...], preferred_element_type=jnp.float32)
            + bo_ref[...]
        )


def _tc_final(parts, W_msg, b_msg, batch3, W_out, b_out):
    return pl.pallas_call(
        _final_body,
        grid=(NGRID,),
        in_specs=[
            pl.BlockSpec((NUM_SC, NB, H), lambda i: (0, i, 0)),
            pl.BlockSpec((H, H), lambda i: (0, 0)),
            pl.BlockSpec((1, H), lambda i: (0, 0)),
            pl.BlockSpec((1, 1, NB), lambda i: (i, 0, 0)),
            pl.BlockSpec((H, OUT), lambda i: (0, 0)),
            pl.BlockSpec((1, OUT), lambda i: (0, 0)),
        ],
        out_specs=pl.BlockSpec((G, OUT), lambda i: (0, 0)),
        out_shape=jax.ShapeDtypeStruct((G, OUT), jnp.float32),
        scratch_shapes=[pltpu.VMEM((G, H), jnp.float32)],
    )(parts, W_msg, b_msg, batch3, W_out, b_out)


def kernel(x, edge_attr, W_node, b_node, W_edge, b_edge, W_msg, b_msg,
           W_out, b_out, edge_index, batch):
    src = edge_index[0]
    dst = edge_index[1]
    e, h = _tc_encode(edge_attr.T, W_edge, b_edge.reshape(1, H),
                      x, W_node, b_node.reshape(1, H))
    zeros = jnp.zeros((N, H), jnp.float32)
    parts = _sc_edge_pass_fn()(h, e, src, dst, zeros)
    return _tc_final(parts, W_msg, b_msg.reshape(1, H),
                     batch.reshape(NGRID, 1, NB), W_out, b_out.reshape(1, OUT))
